# Initial kernel scaffold; baseline (speedup 1.0000x reference)
import jax
import jax.numpy as jnp
from jax.experimental import pallas as pl


def kernel(x0, edge_index0, batch0, x1, edge_index1, batch1, params):
    raise NotImplementedError("write your pallas kernel here")


# speedup vs baseline: 4.6104x; 4.6104x over previous
import functools

import jax
import jax.numpy as jnp
from jax import lax
from jax.experimental import pallas as pl
from jax.experimental.pallas import tpu as pltpu
from jax.experimental.pallas import tpu_sc as plsc

N = 10000
NPAD = 10240
E = 320000
D_IN = 128
DIM = 36
DP = 48
LDIM = 54
ODIM = 18
NG = 128
BN_INV = (1.0 + 1e-5) ** -0.5

CH = 128
CPB = E // CH
TPS = 16
ITERS = (CPB + TPS - 1) // TPS
RPT = NPAD // TPS

BLK = 2000
NB = N // BLK

_mesh = plsc.VectorSubcoreMesh(core_axis_name="c", subcore_axis_name="s")


@functools.partial(
    pl.kernel,
    mesh=_mesh,
    compiler_params=pltpu.CompilerParams(use_tc_tiling_on_sc=False),
    out_type=jax.ShapeDtypeStruct((2 * NPAD, DP), jnp.float32),
    scratch_types=[
        pltpu.VMEM((CH,), jnp.int32),
        pltpu.VMEM((CH,), jnp.int32),
        pltpu.VMEM((CH, DP), jnp.float32),
        pltpu.VMEM_SHARED((2 * NPAD, DP), jnp.float32),
        pltpu.SemaphoreType.DMA,
    ],
)
def _sc_segsum(y_hbm, src_hbm, dst_hbm, zeros_hbm, agg_hbm,
               src_v, dst_v, rows_v, acc_sh, sem):
    c = lax.axis_index("c")
    s = lax.axis_index("s")
    row0 = c * NPAD + s * RPT
    pltpu.sync_copy(zeros_hbm, acc_sh.at[pl.ds(row0, RPT)])
    plsc.subcore_barrier()

    def body(i, carry):
        chunk = s + i * TPS

        @pl.when(chunk < CPB)
        def _():
            base = c * E + chunk * CH
            pltpu.sync_copy(src_hbm.at[pl.ds(base, CH)], src_v)
            pltpu.async_copy(y_hbm.at[src_v], rows_v, sem).wait()
            pltpu.sync_copy(dst_hbm.at[pl.ds(base, CH)], dst_v)
            pltpu.sync_copy(rows_v, acc_sh.at[dst_v], add=True)

        return carry

    lax.fori_loop(0, ITERS, body, 0)
    plsc.subcore_barrier()
    pltpu.sync_copy(acc_sh.at[pl.ds(row0, RPT)], agg_hbm.at[pl.ds(row0, RPT)])


def _proj_body(x_ref, w_ref, y_ref):
    y_ref[...] = jnp.dot(x_ref[0], w_ref[0],
                         preferred_element_type=jnp.float32)[None]


def _proj(xs, w):
    return pl.pallas_call(
        _proj_body,
        grid=(2, NB),
        in_specs=[
            pl.BlockSpec((1, BLK, D_IN), lambda b, i: (b, i, 0)),
            pl.BlockSpec((1, D_IN, DP), lambda b, i: (b, 0, 0)),
        ],
        out_specs=pl.BlockSpec((1, BLK, DP), lambda b, i: (b, i, 0)),
        out_shape=jax.ShapeDtypeStruct((2, NPAD, DP), jnp.float32),
    )(xs, w)


def _round_body(y_ref, agg_ref, wb_ref, wn_ref, ba_ref, bb_ref, g_ref, be_ref,
                out_ref):
    h = jnp.maximum(y_ref[0] + agg_ref[0] + ba_ref[0], 0.0)
    t = jnp.maximum(
        jnp.dot(h, wb_ref[0], preferred_element_type=jnp.float32) + bb_ref[0],
        0.0)
    u = t * (g_ref[0] * BN_INV) + be_ref[0]
    out_ref[...] = jnp.dot(u, wn_ref[0], preferred_element_type=jnp.float32)[None]


def _round(y, agg, wb, wn, ba, bb, g, be):
    vec = pl.BlockSpec((1, 1, DP), lambda b, i: (b, 0, 0))
    mat = pl.BlockSpec((1, DP, DP), lambda b, i: (b, 0, 0))
    row = pl.BlockSpec((1, BLK, DP), lambda b, i: (b, i, 0))
    return pl.pallas_call(
        _round_body,
        grid=(2, NB),
        in_specs=[row, row, mat, mat, vec, vec, vec, vec],
        out_specs=row,
        out_shape=jax.ShapeDtypeStruct((2, NPAD, DP), jnp.float32),
    )(y, agg, wb, wn, ba, bb, g, be)


def _pool_body(y_ref, agg_ref, wb_ref, ba_ref, bb_ref, g_ref, be_ref,
               batch_ref, s_ref, c_ref):
    h = jnp.maximum(y_ref[0] + agg_ref[0] + ba_ref[0], 0.0)
    t = jnp.maximum(
        jnp.dot(h, wb_ref[0], preferred_element_type=jnp.float32) + bb_ref[0],
        0.0)
    z = t * (g_ref[0] * BN_INV) + be_ref[0]
    onehot = (batch_ref[0, 0][:, None] ==
              lax.broadcasted_iota(jnp.int32, (BLK, NG), 1)).astype(jnp.float32)
    spart = lax.dot_general(onehot, z, (((0,), (0,)), ((), ())),
                            preferred_element_type=jnp.float32)
    cpart = jnp.sum(onehot, axis=0)[None]

    @pl.when(pl.program_id(1) == 0)
    def _():
        s_ref[...] = spart[None]
        c_ref[...] = cpart[None]

    @pl.when(pl.program_id(1) > 0)
    def _():
        s_ref[...] += spart[None]
        c_ref[...] += cpart[None]


def _pool(y, agg, wb, ba, bb, g, be, batch_r):
    vec = pl.BlockSpec((1, 1, DP), lambda b, i: (b, 0, 0))
    mat = pl.BlockSpec((1, DP, DP), lambda b, i: (b, 0, 0))
    row = pl.BlockSpec((1, BLK, DP), lambda b, i: (b, i, 0))
    bat = pl.BlockSpec((1, 1, BLK), lambda b, i: (b * NB + i, 0, 0))
    return pl.pallas_call(
        _pool_body,
        grid=(2, NB),
        in_specs=[row, row, mat, vec, vec, vec, vec, bat],
        out_specs=[
            pl.BlockSpec((1, NG, DP), lambda b, i: (b, 0, 0)),
            pl.BlockSpec((1, 1, NG), lambda b, i: (b, 0, 0)),
        ],
        out_shape=[
            jax.ShapeDtypeStruct((2, NG, DP), jnp.float32),
            jax.ShapeDtypeStruct((2, 1, NG), jnp.float32),
        ],
    )(y, agg, wb, ba, bb, g, be, batch_r)


def _head_body(s_ref, c_ref, wm0_ref, wm1_ref, bm_ref, wo_ref, bo_ref,
               wf_ref, bf_ref, o_ref):
    cnt0 = jnp.maximum(c_ref[0, 0], 1.0)
    cnt1 = jnp.maximum(c_ref[1, 0], 1.0)
    e0 = s_ref[0, :, :DIM] / cnt0[:, None]
    e1 = s_ref[1, :, :DIM] / cnt1[:, None]
    h = jnp.maximum(
        jnp.dot(e0, wm0_ref[...], preferred_element_type=jnp.float32)
        + jnp.dot(e1, wm1_ref[...], preferred_element_type=jnp.float32)
        + bm_ref[...], 0.0)
    h = jnp.maximum(
        jnp.dot(h, wo_ref[...], preferred_element_type=jnp.float32)
        + bo_ref[...], 0.0)
    logit = (jnp.dot(h, wf_ref[...], preferred_element_type=jnp.float32)
             + bf_ref[...])
    o_ref[...] = jax.nn.sigmoid(logit)


def _head(s, c, wm0, wm1, bm, wo, bo, wf, bf):
    return pl.pallas_call(
        _head_body,
        out_shape=jax.ShapeDtypeStruct((NG, 1), jnp.float32),
    )(s, c, wm0, wm1, bm, wo, bo, wf, bf)


def _pad_mat(w):
    return jnp.pad(w, ((0, DP - w.shape[0]), (0, DP - w.shape[1])))


def _pad_vec(v):
    return jnp.pad(v, (0, DP - v.shape[0]))


def kernel(x0, edge_index0, batch0, x1, edge_index1, batch1, params):
    p0, p1 = params["t0"], params["t1"]

    def sv(name):
        return jnp.stack([_pad_vec(p0[name]), _pad_vec(p1[name])])[:, None]

    def sm(name):
        return jnp.stack([_pad_mat(p0[name]), _pad_mat(p1[name])])

    w1a = jnp.stack([jnp.pad(p0["W1a"], ((0, 0), (0, DP - DIM))),
                     jnp.pad(p1["W1a"], ((0, 0), (0, DP - DIM)))])
    xs = jnp.stack([x0, x1])
    src = jnp.concatenate([edge_index0[0], edge_index1[0] + NPAD])
    dst = jnp.concatenate([edge_index0[1], edge_index1[1] + NPAD])
    zeros = jnp.zeros((RPT, DP), jnp.float32)
    batch_r = jnp.stack([batch0, batch1]).reshape(2 * NB, 1, BLK)

    y = _proj(xs, w1a)
    agg = _sc_segsum(y.reshape(2 * NPAD, DP), src, dst, zeros).reshape(2, NPAD, DP)
    y = _round(y, agg, sm("W1b"), sm("W2a"), sv("b1a"), sv("b1b"),
               sv("g1"), sv("be1"))
    agg = _sc_segsum(y.reshape(2 * NPAD, DP), src, dst, zeros).reshape(2, NPAD, DP)
    y = _round(y, agg, sm("W2b"), sm("W3a"), sv("b2a"), sv("b2b"),
               sv("g2"), sv("be2"))
    agg = _sc_segsum(y.reshape(2 * NPAD, DP), src, dst, zeros).reshape(2, NPAD, DP)
    s, cnt = _pool(y, agg, sm("W3b"), sv("b3a"), sv("b3b"), sv("g3"),
                   sv("be3"), batch_r)
    wm = params["Wm"]
    return _head(s, cnt, wm[:DIM], wm[DIM:], params["bm"][None],
                 params["Wo"], params["bo"][None], params["Wf"],
                 params["bf"][None])


# speedup vs baseline: 12.0561x; 2.6150x over previous
import functools

import jax
import jax.numpy as jnp
from jax import lax
from jax.experimental import pallas as pl
from jax.experimental.pallas import tpu as pltpu
from jax.experimental.pallas import tpu_sc as plsc

N = 10000
NPAD = 10240
E = 320000
D_IN = 128
DIM = 36
DP = 48
LDIM = 54
ODIM = 18
NG = 128
BN_INV = (1.0 + 1e-5) ** -0.5

CH = 80
TPS = 16
EPT = E // TPS
NCH = EPT // CH
NBUF = 5
NGI = NCH // NBUF
RPT = NPAD // TPS

BLK = 2000
NB = N // BLK

_mesh = plsc.VectorSubcoreMesh(core_axis_name="c", subcore_axis_name="s")


@functools.partial(
    pl.kernel,
    mesh=_mesh,
    compiler_params=pltpu.CompilerParams(use_tc_tiling_on_sc=False),
    out_type=jax.ShapeDtypeStruct((2 * NPAD, DP), jnp.float32),
    scratch_types=[
        pltpu.VMEM((NCH, CH), jnp.int32),
        pltpu.VMEM((NCH, CH), jnp.int32),
        pltpu.VMEM((NBUF, CH, DP), jnp.float32),
        pltpu.VMEM_SHARED((2 * NPAD, DP), jnp.float32),
    ] + [pltpu.SemaphoreType.DMA] * (2 * NBUF),
)
def _sc_segsum(y_hbm, src_hbm, dst_hbm, zeros_hbm, agg_hbm,
               src_v, dst_v, rows_v, acc_sh, *sems):
    sg, ss = sems[:NBUF], sems[NBUF:]
    c = lax.axis_index("c")
    s = lax.axis_index("s")
    row0 = c * NPAD + s * RPT
    tile = c * TPS + s
    pltpu.sync_copy(src_hbm.at[tile], src_v)
    pltpu.sync_copy(dst_hbm.at[tile], dst_v)
    pltpu.sync_copy(zeros_hbm, acc_sh.at[pl.ds(row0, RPT)])
    plsc.subcore_barrier()

    for b in range(NBUF):
        pltpu.async_copy(y_hbm.at[src_v.at[b]], rows_v.at[b], sg[b])

    def body(g, carry):
        jb = g * NBUF
        for b in range(NBUF):
            pltpu.make_async_copy(
                y_hbm.at[src_v.at[jb + b]], rows_v.at[b], sg[b]).wait()
            pltpu.async_copy(
                rows_v.at[b], acc_sh.at[dst_v.at[jb + b]], ss[b], add=True)

        @pl.when(g < NGI - 1)
        def _():
            for b in range(NBUF):
                pltpu.make_async_copy(
                    rows_v.at[b], acc_sh.at[dst_v.at[jb + b]], ss[b]).wait()
                pltpu.async_copy(
                    y_hbm.at[src_v.at[jb + NBUF + b]], rows_v.at[b], sg[b])

        return carry

    lax.fori_loop(0, NGI, body, 0)
    for b in range(NBUF):
        pltpu.make_async_copy(
            rows_v.at[b], acc_sh.at[dst_v.at[(NGI - 1) * NBUF + b]],
            ss[b]).wait()
    plsc.subcore_barrier()
    pltpu.sync_copy(acc_sh.at[pl.ds(row0, RPT)], agg_hbm.at[pl.ds(row0, RPT)])


def _proj_body(x_ref, w_ref, y_ref):
    y_ref[...] = jnp.dot(x_ref[0], w_ref[0],
                         preferred_element_type=jnp.float32)[None]


def _proj(xs, w):
    return pl.pallas_call(
        _proj_body,
        grid=(2, NB),
        in_specs=[
            pl.BlockSpec((1, BLK, D_IN), lambda b, i: (b, i, 0)),
            pl.BlockSpec((1, D_IN, DP), lambda b, i: (b, 0, 0)),
        ],
        out_specs=pl.BlockSpec((1, BLK, DP), lambda b, i: (b, i, 0)),
        out_shape=jax.ShapeDtypeStruct((2, NPAD, DP), jnp.float32),
    )(xs, w)


def _round_body(y_ref, agg_ref, wb_ref, wn_ref, ba_ref, bb_ref, g_ref, be_ref,
                out_ref):
    h = jnp.maximum(y_ref[0] + agg_ref[0] + ba_ref[0], 0.0)
    t = jnp.maximum(
        jnp.dot(h, wb_ref[0], preferred_element_type=jnp.float32) + bb_ref[0],
        0.0)
    u = t * (g_ref[0] * BN_INV) + be_ref[0]
    out_ref[...] = jnp.dot(u, wn_ref[0], preferred_element_type=jnp.float32)[None]


def _round(y, agg, wb, wn, ba, bb, g, be):
    vec = pl.BlockSpec((1, 1, DP), lambda b, i: (b, 0, 0))
    mat = pl.BlockSpec((1, DP, DP), lambda b, i: (b, 0, 0))
    row = pl.BlockSpec((1, BLK, DP), lambda b, i: (b, i, 0))
    return pl.pallas_call(
        _round_body,
        grid=(2, NB),
        in_specs=[row, row, mat, mat, vec, vec, vec, vec],
        out_specs=row,
        out_shape=jax.ShapeDtypeStruct((2, NPAD, DP), jnp.float32),
    )(y, agg, wb, wn, ba, bb, g, be)


def _pool_body(y_ref, agg_ref, wb_ref, ba_ref, bb_ref, g_ref, be_ref,
               batch_ref, s_ref, c_ref):
    h = jnp.maximum(y_ref[0] + agg_ref[0] + ba_ref[0], 0.0)
    t = jnp.maximum(
        jnp.dot(h, wb_ref[0], preferred_element_type=jnp.float32) + bb_ref[0],
        0.0)
    z = t * (g_ref[0] * BN_INV) + be_ref[0]
    onehot = (batch_ref[0, 0][:, None] ==
              lax.broadcasted_iota(jnp.int32, (BLK, NG), 1)).astype(jnp.float32)
    spart = lax.dot_general(onehot, z, (((0,), (0,)), ((), ())),
                            preferred_element_type=jnp.float32)
    cpart = jnp.sum(onehot, axis=0)[None]

    @pl.when(pl.program_id(1) == 0)
    def _():
        s_ref[...] = spart[None]
        c_ref[...] = cpart[None]

    @pl.when(pl.program_id(1) > 0)
    def _():
        s_ref[...] += spart[None]
        c_ref[...] += cpart[None]


def _pool(y, agg, wb, ba, bb, g, be, batch_r):
    vec = pl.BlockSpec((1, 1, DP), lambda b, i: (b, 0, 0))
    mat = pl.BlockSpec((1, DP, DP), lambda b, i: (b, 0, 0))
    row = pl.BlockSpec((1, BLK, DP), lambda b, i: (b, i, 0))
    bat = pl.BlockSpec((1, 1, BLK), lambda b, i: (b * NB + i, 0, 0))
    return pl.pallas_call(
        _pool_body,
        grid=(2, NB),
        in_specs=[row, row, mat, vec, vec, vec, vec, bat],
        out_specs=[
            pl.BlockSpec((1, NG, DP), lambda b, i: (b, 0, 0)),
            pl.BlockSpec((1, 1, NG), lambda b, i: (b, 0, 0)),
        ],
        out_shape=[
            jax.ShapeDtypeStruct((2, NG, DP), jnp.float32),
            jax.ShapeDtypeStruct((2, 1, NG), jnp.float32),
        ],
    )(y, agg, wb, ba, bb, g, be, batch_r)


def _head_body(s_ref, c_ref, wm0_ref, wm1_ref, bm_ref, wo_ref, bo_ref,
               wf_ref, bf_ref, o_ref):
    cnt0 = jnp.maximum(c_ref[0, 0], 1.0)
    cnt1 = jnp.maximum(c_ref[1, 0], 1.0)
    e0 = s_ref[0, :, :DIM] / cnt0[:, None]
    e1 = s_ref[1, :, :DIM] / cnt1[:, None]
    h = jnp.maximum(
        jnp.dot(e0, wm0_ref[...], preferred_element_type=jnp.float32)
        + jnp.dot(e1, wm1_ref[...], preferred_element_type=jnp.float32)
        + bm_ref[...], 0.0)
    h = jnp.maximum(
        jnp.dot(h, wo_ref[...], preferred_element_type=jnp.float32)
        + bo_ref[...], 0.0)
    logit = (jnp.dot(h, wf_ref[...], preferred_element_type=jnp.float32)
             + bf_ref[...])
    o_ref[...] = jax.nn.sigmoid(logit)


def _head(s, c, wm0, wm1, bm, wo, bo, wf, bf):
    return pl.pallas_call(
        _head_body,
        out_shape=jax.ShapeDtypeStruct((NG, 1), jnp.float32),
    )(s, c, wm0, wm1, bm, wo, bo, wf, bf)


def _pad_mat(w):
    return jnp.pad(w, ((0, DP - w.shape[0]), (0, DP - w.shape[1])))


def _pad_vec(v):
    return jnp.pad(v, (0, DP - v.shape[0]))


def kernel(x0, edge_index0, batch0, x1, edge_index1, batch1, params):
    p0, p1 = params["t0"], params["t1"]

    def sv(name):
        return jnp.stack([_pad_vec(p0[name]), _pad_vec(p1[name])])[:, None]

    def sm(name):
        return jnp.stack([_pad_mat(p0[name]), _pad_mat(p1[name])])

    w1a = jnp.stack([jnp.pad(p0["W1a"], ((0, 0), (0, DP - DIM))),
                     jnp.pad(p1["W1a"], ((0, 0), (0, DP - DIM)))])
    xs = jnp.stack([x0, x1])
    src = jnp.concatenate([edge_index0[0], edge_index1[0] + NPAD]).reshape(
        2 * TPS, NCH, CH)
    dst = jnp.concatenate([edge_index0[1], edge_index1[1] + NPAD]).reshape(
        2 * TPS, NCH, CH)
    zeros = jnp.zeros((RPT, DP), jnp.float32)
    batch_r = jnp.stack([batch0, batch1]).reshape(2 * NB, 1, BLK)

    y = _proj(xs, w1a)
    agg = _sc_segsum(y.reshape(2 * NPAD, DP), src, dst, zeros).reshape(2, NPAD, DP)
    y = _round(y, agg, sm("W1b"), sm("W2a"), sv("b1a"), sv("b1b"),
               sv("g1"), sv("be1"))
    agg = _sc_segsum(y.reshape(2 * NPAD, DP), src, dst, zeros).reshape(2, NPAD, DP)
    y = _round(y, agg, sm("W2b"), sm("W3a"), sv("b2a"), sv("b2b"),
               sv("g2"), sv("be2"))
    agg = _sc_segsum(y.reshape(2 * NPAD, DP), src, dst, zeros).reshape(2, NPAD, DP)
    s, cnt = _pool(y, agg, sm("W3b"), sv("b3a"), sv("b3b"), sv("g3"),
                   sv("be3"), batch_r)
    wm = params["Wm"]
    return _head(s, cnt, wm[:DIM], wm[DIM:], params["bm"][None],
                 params["Wo"], params["bo"][None], params["Wf"],
                 params["bf"][None])


# speedup vs baseline: 12.3628x; 1.0254x over previous
import functools

import jax
import jax.numpy as jnp
from jax import lax
from jax.experimental import pallas as pl
from jax.experimental.pallas import tpu as pltpu
from jax.experimental.pallas import tpu_sc as plsc

N = 10000
NPAD = 10240
E = 320000
D_IN = 128
DIM = 36
DP = 36
NG = 128
BN_INV = (1.0 + 1e-5) ** -0.5

CH = 80
TPS = 16
EPT = E // TPS
NCH = EPT // CH
NBUF = 5
NGI = NCH // NBUF
RPT = NPAD // TPS

BLK = 2000
NB = N // BLK

_mesh = plsc.VectorSubcoreMesh(core_axis_name="c", subcore_axis_name="s")


@functools.partial(
    pl.kernel,
    mesh=_mesh,
    compiler_params=pltpu.CompilerParams(use_tc_tiling_on_sc=False),
    out_type=jax.ShapeDtypeStruct((2 * NPAD, DP), jnp.float32),
    scratch_types=[
        pltpu.VMEM((NCH, CH), jnp.int32),
        pltpu.VMEM((NCH, CH), jnp.int32),
        pltpu.VMEM((NBUF, CH, DP), jnp.float32),
        pltpu.VMEM_SHARED((2 * NPAD, DP), jnp.float32),
    ] + [pltpu.SemaphoreType.DMA] * (2 * NBUF),
)
def _sc_segsum(y_hbm, src_hbm, dst_hbm, zeros_hbm, agg_hbm,
               src_v, dst_v, rows_v, acc_sh, *sems):
    sg, ss = sems[:NBUF], sems[NBUF:]
    c = lax.axis_index("c")
    s = lax.axis_index("s")
    row0 = c * NPAD + s * RPT
    tile = c * TPS + s
    pltpu.sync_copy(src_hbm.at[tile], src_v)
    pltpu.sync_copy(dst_hbm.at[tile], dst_v)
    pltpu.sync_copy(zeros_hbm, acc_sh.at[pl.ds(row0, RPT)])
    plsc.subcore_barrier()

    for b in range(NBUF):
        pltpu.async_copy(y_hbm.at[src_v.at[b]], rows_v.at[b], sg[b])

    def body(g, carry):
        jb = g * NBUF
        for b in range(NBUF):
            pltpu.make_async_copy(
                y_hbm.at[src_v.at[jb + b]], rows_v.at[b], sg[b]).wait()
            pltpu.async_copy(
                rows_v.at[b], acc_sh.at[dst_v.at[jb + b]], ss[b], add=True)

        @pl.when(g < NGI - 1)
        def _():
            for b in range(NBUF):
                pltpu.make_async_copy(
                    rows_v.at[b], acc_sh.at[dst_v.at[jb + b]], ss[b]).wait()
                pltpu.async_copy(
                    y_hbm.at[src_v.at[jb + NBUF + b]], rows_v.at[b], sg[b])

        return carry

    lax.fori_loop(0, NGI, body, 0)
    for b in range(NBUF):
        pltpu.make_async_copy(
            rows_v.at[b], acc_sh.at[dst_v.at[(NGI - 1) * NBUF + b]],
            ss[b]).wait()
    plsc.subcore_barrier()
    pltpu.sync_copy(acc_sh.at[pl.ds(row0, RPT)], agg_hbm.at[pl.ds(row0, RPT)])


def _proj_body(x0_ref, x1_ref, w0_ref, w1_ref, y_ref):
    y_ref[0] = jnp.dot(x0_ref[...], w0_ref[...],
                       preferred_element_type=jnp.float32)
    y_ref[1] = jnp.dot(x1_ref[...], w1_ref[...],
                       preferred_element_type=jnp.float32)


def _proj(x0, x1, w0, w1):
    return pl.pallas_call(
        _proj_body,
        grid=(NB,),
        in_specs=[
            pl.BlockSpec((BLK, D_IN), lambda i: (i, 0)),
            pl.BlockSpec((BLK, D_IN), lambda i: (i, 0)),
            pl.BlockSpec((D_IN, DIM), lambda i: (0, 0)),
            pl.BlockSpec((D_IN, DIM), lambda i: (0, 0)),
        ],
        out_specs=pl.BlockSpec((2, BLK, DP), lambda i: (0, i, 0)),
        out_shape=jax.ShapeDtypeStruct((2, NPAD, DP), jnp.float32),
    )(x0, x1, w0, w1)


def _layer(y, agg, wb, ba, bb, g, be):
    h = jnp.maximum(y + agg + ba, 0.0)
    t = jnp.maximum(
        jnp.dot(h, wb, preferred_element_type=jnp.float32) + bb, 0.0)
    return t * (g * BN_INV) + be


def _round_body(y_ref, agg_ref, wb0, wn0, ba0, bb0, g0, be0,
                wb1, wn1, ba1, bb1, g1, be1, out_ref):
    u0 = _layer(y_ref[0], agg_ref[0], wb0[...], ba0[...], bb0[...],
                g0[...], be0[...])
    out_ref[0] = jnp.dot(u0, wn0[...], preferred_element_type=jnp.float32)
    u1 = _layer(y_ref[1], agg_ref[1], wb1[...], ba1[...], bb1[...],
                g1[...], be1[...])
    out_ref[1] = jnp.dot(u1, wn1[...], preferred_element_type=jnp.float32)


def _round(y, agg, w0, w1):
    row = pl.BlockSpec((2, BLK, DP), lambda i: (0, i, 0))
    mat = pl.BlockSpec((DIM, DIM), lambda i: (0, 0))
    vec = pl.BlockSpec((DIM,), lambda i: (0,))
    return pl.pallas_call(
        _round_body,
        grid=(NB,),
        in_specs=[row, row] + [mat, mat, vec, vec, vec, vec] * 2,
        out_specs=row,
        out_shape=jax.ShapeDtypeStruct((2, NPAD, DP), jnp.float32),
    )(y, agg, *w0, *w1)


def _pool_body(y_ref, agg_ref, wb0, ba0, bb0, g0, be0,
               wb1, ba1, bb1, g1, be1, bat0, bat1, s_ref, c_ref):
    i = pl.program_id(0)
    for b, (wb, ba, bb, g, be, bat) in enumerate(
            ((wb0, ba0, bb0, g0, be0, bat0), (wb1, ba1, bb1, g1, be1, bat1))):
        z = _layer(y_ref[b], agg_ref[b], wb[...], ba[...], bb[...],
                   g[...], be[...])
        onehot = (bat[0, 0][:, None] ==
                  lax.broadcasted_iota(jnp.int32, (BLK, NG), 1)
                  ).astype(jnp.float32)
        spart = lax.dot_general(onehot, z, (((0,), (0,)), ((), ())),
                                preferred_element_type=jnp.float32)
        cpart = jnp.sum(onehot, axis=0)[None]

        @pl.when(i == 0)
        def _(b=b, spart=spart, cpart=cpart):
            s_ref[b] = spart
            c_ref[b] = cpart

        @pl.when(i > 0)
        def _(b=b, spart=spart, cpart=cpart):
            s_ref[b] += spart
            c_ref[b] += cpart


def _pool(y, agg, w0, w1, bat0, bat1):
    row = pl.BlockSpec((2, BLK, DP), lambda i: (0, i, 0))
    mat = pl.BlockSpec((DIM, DIM), lambda i: (0, 0))
    vec = pl.BlockSpec((DIM,), lambda i: (0,))
    bat = pl.BlockSpec((1, 1, BLK), lambda i: (i, 0, 0))
    return pl.pallas_call(
        _pool_body,
        grid=(NB,),
        in_specs=[row, row] + [mat, vec, vec, vec, vec] * 2 + [bat, bat],
        out_specs=[
            pl.BlockSpec((2, NG, DP), lambda i: (0, 0, 0)),
            pl.BlockSpec((2, 1, NG), lambda i: (0, 0, 0)),
        ],
        out_shape=[
            jax.ShapeDtypeStruct((2, NG, DP), jnp.float32),
            jax.ShapeDtypeStruct((2, 1, NG), jnp.float32),
        ],
    )(y, agg, *w0, *w1, bat0, bat1)


def _head_body(s_ref, c_ref, wm0_ref, wm1_ref, bm_ref, wo_ref, bo_ref,
               wf_ref, bf_ref, o_ref):
    cnt0 = jnp.maximum(c_ref[0, 0], 1.0)
    cnt1 = jnp.maximum(c_ref[1, 0], 1.0)
    e0 = s_ref[0] / cnt0[:, None]
    e1 = s_ref[1] / cnt1[:, None]
    h = jnp.maximum(
        jnp.dot(e0, wm0_ref[...], preferred_element_type=jnp.float32)
        + jnp.dot(e1, wm1_ref[...], preferred_element_type=jnp.float32)
        + bm_ref[...], 0.0)
    h = jnp.maximum(
        jnp.dot(h, wo_ref[...], preferred_element_type=jnp.float32)
        + bo_ref[...], 0.0)
    logit = (jnp.dot(h, wf_ref[...], preferred_element_type=jnp.float32)
             + bf_ref[...])
    o_ref[...] = jax.nn.sigmoid(logit)


def _head(s, c, wm0, wm1, bm, wo, bo, wf, bf):
    return pl.pallas_call(
        _head_body,
        out_shape=jax.ShapeDtypeStruct((NG, 1), jnp.float32),
    )(s, c, wm0, wm1, bm, wo, bo, wf, bf)


def kernel(x0, edge_index0, batch0, x1, edge_index1, batch1, params):
    p0, p1 = params["t0"], params["t1"]
    src = jnp.concatenate([edge_index0[0], edge_index1[0] + NPAD]).reshape(
        2 * TPS, NCH, CH)
    dst = jnp.concatenate([edge_index0[1], edge_index1[1] + NPAD]).reshape(
        2 * TPS, NCH, CH)
    zeros = jnp.zeros((RPT, DP), jnp.float32)
    bat0 = batch0.reshape(NB, 1, BLK)
    bat1 = batch1.reshape(NB, 1, BLK)

    def rw(p, r):
        return (p["W%db" % r], p["W%da" % (r + 1)], p["b%da" % r],
                p["b%db" % r], p["g%d" % r], p["be%d" % r])

    def pw(p):
        return (p["W3b"], p["b3a"], p["b3b"], p["g3"], p["be3"])

    y = _proj(x0, x1, p0["W1a"], p1["W1a"])
    agg = _sc_segsum(y.reshape(2 * NPAD, DP), src, dst,
                     zeros).reshape(2, NPAD, DP)
    y = _round(y, agg, rw(p0, 1), rw(p1, 1))
    agg = _sc_segsum(y.reshape(2 * NPAD, DP), src, dst,
                     zeros).reshape(2, NPAD, DP)
    y = _round(y, agg, rw(p0, 2), rw(p1, 2))
    agg = _sc_segsum(y.reshape(2 * NPAD, DP), src, dst,
                     zeros).reshape(2, NPAD, DP)
    s, cnt = _pool(y, agg, pw(p0), pw(p1), bat0, bat1)
    wm = params["Wm"]
    return _head(s, cnt, wm[:DIM], wm[DIM:], params["bm"][None],
                 params["Wo"], params["bo"][None], params["Wf"],
                 params["bf"][None])


# speedup vs baseline: 12.5281x; 1.0134x over previous
import functools

import jax
import jax.numpy as jnp
from jax import lax
from jax.experimental import pallas as pl
from jax.experimental.pallas import tpu as pltpu
from jax.experimental.pallas import tpu_sc as plsc

N = 10000
NPAD = 10240
E = 320000
D_IN = 128
DIM = 36
DP = 48
NG = 128
BN_INV = (1.0 + 1e-5) ** -0.5

CH = 80
TPS = 16
EPT = E // TPS
NCH = EPT // CH
NBUF = 5
NGI = NCH // NBUF
RPT = NPAD // TPS

BLK = 2000
NB = N // BLK

_mesh = plsc.VectorSubcoreMesh(core_axis_name="c", subcore_axis_name="s")


@functools.partial(
    pl.kernel,
    mesh=_mesh,
    compiler_params=pltpu.CompilerParams(use_tc_tiling_on_sc=False),
    out_type=jax.ShapeDtypeStruct((2 * NPAD, DP), jnp.float32),
    scratch_types=[
        pltpu.VMEM((NCH, CH), jnp.int32),
        pltpu.VMEM((NCH, CH), jnp.int32),
        pltpu.VMEM((NBUF, CH, DP), jnp.float32),
        pltpu.VMEM_SHARED((2 * NPAD, DP), jnp.float32),
    ] + [pltpu.SemaphoreType.DMA] * (2 * NBUF),
)
def _sc_segsum(y_hbm, src_hbm, dst_hbm, zeros_hbm, agg_hbm,
               src_v, dst_v, rows_v, acc_sh, *sems):
    sg, ss = sems[:NBUF], sems[NBUF:]
    c = lax.axis_index("c")
    s = lax.axis_index("s")
    row0 = c * NPAD + s * RPT
    tile = c * TPS + s
    pltpu.sync_copy(src_hbm.at[tile], src_v)
    pltpu.sync_copy(dst_hbm.at[tile], dst_v)
    pltpu.sync_copy(zeros_hbm, acc_sh.at[pl.ds(row0, RPT)])
    plsc.subcore_barrier()

    for b in range(NBUF):
        pltpu.async_copy(y_hbm.at[src_v.at[b]], rows_v.at[b], sg[b])

    def body(g, carry):
        jb = g * NBUF
        for b in range(NBUF):
            pltpu.make_async_copy(
                y_hbm.at[src_v.at[jb + b]], rows_v.at[b], sg[b]).wait()
            pltpu.async_copy(
                rows_v.at[b], acc_sh.at[dst_v.at[jb + b]], ss[b], add=True)

        @pl.when(g < NGI - 1)
        def _():
            for b in range(NBUF):
                pltpu.make_async_copy(
                    rows_v.at[b], acc_sh.at[dst_v.at[jb + b]], ss[b]).wait()
                pltpu.async_copy(
                    y_hbm.at[src_v.at[jb + NBUF + b]], rows_v.at[b], sg[b])

        return carry

    lax.fori_loop(0, NGI, body, 0)
    for b in range(NBUF):
        pltpu.make_async_copy(
            rows_v.at[b], acc_sh.at[dst_v.at[(NGI - 1) * NBUF + b]],
            ss[b]).wait()
    plsc.subcore_barrier()
    pltpu.sync_copy(acc_sh.at[pl.ds(row0, RPT)], agg_hbm.at[pl.ds(row0, RPT)])


def _padw(t):
    return jnp.pad(t, ((0, 0), (0, DP - DIM)))


def _proj_body(x0_ref, x1_ref, w0_ref, w1_ref, y_ref):
    y_ref[0] = _padw(jnp.dot(x0_ref[...], w0_ref[...],
                             preferred_element_type=jnp.float32))
    y_ref[1] = _padw(jnp.dot(x1_ref[...], w1_ref[...],
                             preferred_element_type=jnp.float32))


def _proj(x0, x1, w0, w1):
    return pl.pallas_call(
        _proj_body,
        grid=(NB,),
        in_specs=[
            pl.BlockSpec((BLK, D_IN), lambda i: (i, 0)),
            pl.BlockSpec((BLK, D_IN), lambda i: (i, 0)),
            pl.BlockSpec((D_IN, DIM), lambda i: (0, 0)),
            pl.BlockSpec((D_IN, DIM), lambda i: (0, 0)),
        ],
        out_specs=pl.BlockSpec((2, BLK, DP), lambda i: (0, i, 0)),
        out_shape=jax.ShapeDtypeStruct((2, NPAD, DP), jnp.float32),
    )(x0, x1, w0, w1)


def _layer(y, agg, wb, ba, bb, g, be):
    h = jnp.maximum((y + agg)[:, :DIM] + ba, 0.0)
    t = jnp.maximum(
        jnp.dot(h, wb, preferred_element_type=jnp.float32) + bb, 0.0)
    return t * (g * BN_INV) + be


def _round_body(y_ref, agg_ref, wb0, wn0, ba0, bb0, g0, be0,
                wb1, wn1, ba1, bb1, g1, be1, out_ref):
    u0 = _layer(y_ref[0], agg_ref[0], wb0[...], ba0[...], bb0[...],
                g0[...], be0[...])
    out_ref[0] = _padw(jnp.dot(u0, wn0[...], preferred_element_type=jnp.float32))
    u1 = _layer(y_ref[1], agg_ref[1], wb1[...], ba1[...], bb1[...],
                g1[...], be1[...])
    out_ref[1] = _padw(jnp.dot(u1, wn1[...], preferred_element_type=jnp.float32))


def _round(y, agg, w0, w1):
    row = pl.BlockSpec((2, BLK, DP), lambda i: (0, i, 0))
    mat = pl.BlockSpec((DIM, DIM), lambda i: (0, 0))
    vec = pl.BlockSpec((DIM,), lambda i: (0,))
    return pl.pallas_call(
        _round_body,
        grid=(NB,),
        in_specs=[row, row] + [mat, mat, vec, vec, vec, vec] * 2,
        out_specs=row,
        out_shape=jax.ShapeDtypeStruct((2, NPAD, DP), jnp.float32),
    )(y, agg, *w0, *w1)


def _pool_body(y_ref, agg_ref, wb0, ba0, bb0, g0, be0,
               wb1, ba1, bb1, g1, be1, bat0, bat1, s_ref, c_ref):
    i = pl.program_id(0)
    for b, (wb, ba, bb, g, be, bat) in enumerate(
            ((wb0, ba0, bb0, g0, be0, bat0), (wb1, ba1, bb1, g1, be1, bat1))):
        z = _layer(y_ref[b], agg_ref[b], wb[...], ba[...], bb[...],
                   g[...], be[...])
        onehot = (bat[0, 0][:, None] ==
                  lax.broadcasted_iota(jnp.int32, (BLK, NG), 1)
                  ).astype(jnp.float32)
        spart = lax.dot_general(onehot, z, (((0,), (0,)), ((), ())),
                                preferred_element_type=jnp.float32)
        cpart = jnp.sum(onehot, axis=0)[None]

        @pl.when(i == 0)
        def _(b=b, spart=spart, cpart=cpart):
            s_ref[b] = spart
            c_ref[b] = cpart

        @pl.when(i > 0)
        def _(b=b, spart=spart, cpart=cpart):
            s_ref[b] += spart
            c_ref[b] += cpart


def _pool(y, agg, w0, w1, bat0, bat1):
    row = pl.BlockSpec((2, BLK, DP), lambda i: (0, i, 0))
    mat = pl.BlockSpec((DIM, DIM), lambda i: (0, 0))
    vec = pl.BlockSpec((DIM,), lambda i: (0,))
    bat = pl.BlockSpec((1, 1, BLK), lambda i: (i, 0, 0))
    return pl.pallas_call(
        _pool_body,
        grid=(NB,),
        in_specs=[row, row] + [mat, vec, vec, vec, vec] * 2 + [bat, bat],
        out_specs=[
            pl.BlockSpec((2, NG, DIM), lambda i: (0, 0, 0)),
            pl.BlockSpec((2, 1, NG), lambda i: (0, 0, 0)),
        ],
        out_shape=[
            jax.ShapeDtypeStruct((2, NG, DIM), jnp.float32),
            jax.ShapeDtypeStruct((2, 1, NG), jnp.float32),
        ],
    )(y, agg, *w0, *w1, bat0, bat1)


def _head_body(s_ref, c_ref, wm0_ref, wm1_ref, bm_ref, wo_ref, bo_ref,
               wf_ref, bf_ref, o_ref):
    cnt0 = jnp.maximum(c_ref[0, 0], 1.0)
    cnt1 = jnp.maximum(c_ref[1, 0], 1.0)
    e0 = s_ref[0] / cnt0[:, None]
    e1 = s_ref[1] / cnt1[:, None]
    h = jnp.maximum(
        jnp.dot(e0, wm0_ref[...], preferred_element_type=jnp.float32)
        + jnp.dot(e1, wm1_ref[...], preferred_element_type=jnp.float32)
        + bm_ref[...], 0.0)
    h = jnp.maximum(
        jnp.dot(h, wo_ref[...], preferred_element_type=jnp.float32)
        + bo_ref[...], 0.0)
    logit = (jnp.dot(h, wf_ref[...], preferred_element_type=jnp.float32)
             + bf_ref[...])
    o_ref[...] = jax.nn.sigmoid(logit)


def _head(s, c, wm0, wm1, bm, wo, bo, wf, bf):
    return pl.pallas_call(
        _head_body,
        out_shape=jax.ShapeDtypeStruct((NG, 1), jnp.float32),
    )(s, c, wm0, wm1, bm, wo, bo, wf, bf)


def kernel(x0, edge_index0, batch0, x1, edge_index1, batch1, params):
    p0, p1 = params["t0"], params["t1"]
    src = jnp.concatenate([edge_index0[0], edge_index1[0] + NPAD]).reshape(
        2 * TPS, NCH, CH)
    dst = jnp.concatenate([edge_index0[1], edge_index1[1] + NPAD]).reshape(
        2 * TPS, NCH, CH)
    zeros = jnp.zeros((RPT, DP), jnp.float32)
    bat0 = batch0.reshape(NB, 1, BLK)
    bat1 = batch1.reshape(NB, 1, BLK)

    def rw(p, r):
        return (p["W%db" % r], p["W%da" % (r + 1)], p["b%da" % r],
                p["b%db" % r], p["g%d" % r], p["be%d" % r])

    def pw(p):
        return (p["W3b"], p["b3a"], p["b3b"], p["g3"], p["be3"])

    y = _proj(x0, x1, p0["W1a"], p1["W1a"])
    agg = _sc_segsum(y.reshape(2 * NPAD, DP), src, dst,
                     zeros).reshape(2, NPAD, DP)
    y = _round(y, agg, rw(p0, 1), rw(p1, 1))
    agg = _sc_segsum(y.reshape(2 * NPAD, DP), src, dst,
                     zeros).reshape(2, NPAD, DP)
    y = _round(y, agg, rw(p0, 2), rw(p1, 2))
    agg = _sc_segsum(y.reshape(2 * NPAD, DP), src, dst,
                     zeros).reshape(2, NPAD, DP)
    s, cnt = _pool(y, agg, pw(p0), pw(p1), bat0, bat1)
    wm = params["Wm"]
    return _head(s, cnt, wm[:DIM], wm[DIM:], params["bm"][None],
                 params["Wo"], params["bo"][None], params["Wf"],
                 params["bf"][None])


# speedup vs baseline: 12.9197x; 1.0313x over previous
import functools

import jax
import jax.numpy as jnp
from jax import lax
from jax.experimental import pallas as pl
from jax.experimental.pallas import tpu as pltpu
from jax.experimental.pallas import tpu_sc as plsc

N = 10000
NPAD = 10240
E = 320000
D_IN = 128
DIM = 36
DP = 48
NG = 128
BN_INV = (1.0 + 1e-5) ** -0.5

CH = 80
TPS = 16
EPT = E // TPS
NCH = EPT // CH
NBUF = 10
NGI = NCH // NBUF
RPT = NPAD // TPS

BLK = 2000
NB = N // BLK

_mesh = plsc.VectorSubcoreMesh(core_axis_name="c", subcore_axis_name="s")


@functools.partial(
    pl.kernel,
    mesh=_mesh,
    compiler_params=pltpu.CompilerParams(use_tc_tiling_on_sc=False),
    out_type=jax.ShapeDtypeStruct((2 * NPAD, DP), jnp.float32),
    scratch_types=[
        pltpu.VMEM((NCH, CH), jnp.int32),
        pltpu.VMEM((NCH, CH), jnp.int32),
        pltpu.VMEM((NBUF, CH, DP), jnp.float32),
        pltpu.VMEM_SHARED((NPAD, DP), jnp.float32),
    ] + [pltpu.SemaphoreType.DMA] * (2 * NBUF),
)
def _sc_segsum(y_hbm, src_hbm, dst_hbm, zeros_hbm, agg_hbm,
               src_v, dst_v, rows_v, acc_sh, *sems):
    sg, ss = sems[:NBUF], sems[NBUF:]
    c = lax.axis_index("c")
    s = lax.axis_index("s")
    row0 = s * RPT
    out0 = c * NPAD + s * RPT
    tile = c * TPS + s
    pltpu.sync_copy(src_hbm.at[tile], src_v)
    pltpu.sync_copy(dst_hbm.at[tile], dst_v)
    pltpu.sync_copy(zeros_hbm, acc_sh.at[pl.ds(row0, RPT)])
    plsc.subcore_barrier()

    for b in range(NBUF):
        pltpu.async_copy(y_hbm.at[src_v.at[b]], rows_v.at[b], sg[b])

    def body(g, carry):
        jb = g * NBUF
        for b in range(NBUF):
            pltpu.make_async_copy(
                y_hbm.at[src_v.at[jb + b]], rows_v.at[b], sg[b]).wait()
            pltpu.async_copy(
                rows_v.at[b], acc_sh.at[dst_v.at[jb + b]], ss[b], add=True)

        @pl.when(g < NGI - 1)
        def _():
            for b in range(NBUF):
                pltpu.make_async_copy(
                    rows_v.at[b], acc_sh.at[dst_v.at[jb + b]], ss[b]).wait()
                pltpu.async_copy(
                    y_hbm.at[src_v.at[jb + NBUF + b]], rows_v.at[b], sg[b])

        return carry

    lax.fori_loop(0, NGI, body, 0)
    for b in range(NBUF):
        pltpu.make_async_copy(
            rows_v.at[b], acc_sh.at[dst_v.at[(NGI - 1) * NBUF + b]],
            ss[b]).wait()
    plsc.subcore_barrier()
    pltpu.sync_copy(acc_sh.at[pl.ds(row0, RPT)], agg_hbm.at[pl.ds(out0, RPT)])


def _padw(t):
    return jnp.pad(t, ((0, 0), (0, DP - DIM)))


def _proj_body(x0_ref, x1_ref, w0_ref, w1_ref, y_ref):
    y_ref[0] = _padw(jnp.dot(x0_ref[...], w0_ref[...],
                             preferred_element_type=jnp.float32))
    y_ref[1] = _padw(jnp.dot(x1_ref[...], w1_ref[...],
                             preferred_element_type=jnp.float32))


def _proj(x0, x1, w0, w1):
    return pl.pallas_call(
        _proj_body,
        grid=(NB,),
        in_specs=[
            pl.BlockSpec((BLK, D_IN), lambda i: (i, 0)),
            pl.BlockSpec((BLK, D_IN), lambda i: (i, 0)),
            pl.BlockSpec((D_IN, DIM), lambda i: (0, 0)),
            pl.BlockSpec((D_IN, DIM), lambda i: (0, 0)),
        ],
        out_specs=pl.BlockSpec((2, BLK, DP), lambda i: (0, i, 0)),
        out_shape=jax.ShapeDtypeStruct((2, NPAD, DP), jnp.float32),
    )(x0, x1, w0, w1)


def _layer(y, agg, wb, ba, bb, g, be):
    h = jnp.maximum((y + agg)[:, :DIM] + ba, 0.0)
    t = jnp.maximum(
        jnp.dot(h, wb, preferred_element_type=jnp.float32) + bb, 0.0)
    return t * (g * BN_INV) + be


def _round_body(y_ref, agg_ref, wb0, wn0, ba0, bb0, g0, be0,
                wb1, wn1, ba1, bb1, g1, be1, out_ref):
    u0 = _layer(y_ref[0], agg_ref[0], wb0[...], ba0[...], bb0[...],
                g0[...], be0[...])
    out_ref[0] = _padw(jnp.dot(u0, wn0[...], preferred_element_type=jnp.float32))
    u1 = _layer(y_ref[1], agg_ref[1], wb1[...], ba1[...], bb1[...],
                g1[...], be1[...])
    out_ref[1] = _padw(jnp.dot(u1, wn1[...], preferred_element_type=jnp.float32))


def _round(y, agg, w0, w1):
    row = pl.BlockSpec((2, BLK, DP), lambda i: (0, i, 0))
    mat = pl.BlockSpec((DIM, DIM), lambda i: (0, 0))
    vec = pl.BlockSpec((DIM,), lambda i: (0,))
    return pl.pallas_call(
        _round_body,
        grid=(NB,),
        in_specs=[row, row] + [mat, mat, vec, vec, vec, vec] * 2,
        out_specs=row,
        out_shape=jax.ShapeDtypeStruct((2, NPAD, DP), jnp.float32),
    )(y, agg, *w0, *w1)


def _pool_body(y_ref, agg_ref, wb0, ba0, bb0, g0, be0,
               wb1, ba1, bb1, g1, be1, bat0, bat1, s_ref, c_ref):
    i = pl.program_id(0)
    for b, (wb, ba, bb, g, be, bat) in enumerate(
            ((wb0, ba0, bb0, g0, be0, bat0), (wb1, ba1, bb1, g1, be1, bat1))):
        z = _layer(y_ref[b], agg_ref[b], wb[...], ba[...], bb[...],
                   g[...], be[...])
        onehot = (bat[0, 0][:, None] ==
                  lax.broadcasted_iota(jnp.int32, (BLK, NG), 1)
                  ).astype(jnp.float32)
        spart = lax.dot_general(onehot, z, (((0,), (0,)), ((), ())),
                                preferred_element_type=jnp.float32)
        cpart = jnp.sum(onehot, axis=0)[None]

        @pl.when(i == 0)
        def _(b=b, spart=spart, cpart=cpart):
            s_ref[b] = spart
            c_ref[b] = cpart

        @pl.when(i > 0)
        def _(b=b, spart=spart, cpart=cpart):
            s_ref[b] += spart
            c_ref[b] += cpart


def _pool(y, agg, w0, w1, bat0, bat1):
    row = pl.BlockSpec((2, BLK, DP), lambda i: (0, i, 0))
    mat = pl.BlockSpec((DIM, DIM), lambda i: (0, 0))
    vec = pl.BlockSpec((DIM,), lambda i: (0,))
    bat = pl.BlockSpec((1, 1, BLK), lambda i: (i, 0, 0))
    return pl.pallas_call(
        _pool_body,
        grid=(NB,),
        in_specs=[row, row] + [mat, vec, vec, vec, vec] * 2 + [bat, bat],
        out_specs=[
            pl.BlockSpec((2, NG, DIM), lambda i: (0, 0, 0)),
            pl.BlockSpec((2, 1, NG), lambda i: (0, 0, 0)),
        ],
        out_shape=[
            jax.ShapeDtypeStruct((2, NG, DIM), jnp.float32),
            jax.ShapeDtypeStruct((2, 1, NG), jnp.float32),
        ],
    )(y, agg, *w0, *w1, bat0, bat1)


def _head_body(s_ref, c_ref, wm0_ref, wm1_ref, bm_ref, wo_ref, bo_ref,
               wf_ref, bf_ref, o_ref):
    cnt0 = jnp.maximum(c_ref[0, 0], 1.0)
    cnt1 = jnp.maximum(c_ref[1, 0], 1.0)
    e0 = s_ref[0] / cnt0[:, None]
    e1 = s_ref[1] / cnt1[:, None]
    h = jnp.maximum(
        jnp.dot(e0, wm0_ref[...], preferred_element_type=jnp.float32)
        + jnp.dot(e1, wm1_ref[...], preferred_element_type=jnp.float32)
        + bm_ref[...], 0.0)
    h = jnp.maximum(
        jnp.dot(h, wo_ref[...], preferred_element_type=jnp.float32)
        + bo_ref[...], 0.0)
    logit = (jnp.dot(h, wf_ref[...], preferred_element_type=jnp.float32)
             + bf_ref[...])
    o_ref[...] = jax.nn.sigmoid(logit)


def _head(s, c, wm0, wm1, bm, wo, bo, wf, bf):
    return pl.pallas_call(
        _head_body,
        out_shape=jax.ShapeDtypeStruct((NG, 1), jnp.float32),
    )(s, c, wm0, wm1, bm, wo, bo, wf, bf)


def kernel(x0, edge_index0, batch0, x1, edge_index1, batch1, params):
    p0, p1 = params["t0"], params["t1"]
    src = jnp.concatenate([edge_index0[0], edge_index1[0] + NPAD]).reshape(
        2 * TPS, NCH, CH)
    dst = jnp.concatenate([edge_index0[1], edge_index1[1]]).reshape(
        2 * TPS, NCH, CH)
    zeros = jnp.zeros((RPT, DP), jnp.float32)
    bat0 = batch0.reshape(NB, 1, BLK)
    bat1 = batch1.reshape(NB, 1, BLK)

    def rw(p, r):
        return (p["W%db" % r], p["W%da" % (r + 1)], p["b%da" % r],
                p["b%db" % r], p["g%d" % r], p["be%d" % r])

    def pw(p):
        return (p["W3b"], p["b3a"], p["b3b"], p["g3"], p["be3"])

    y = _proj(x0, x1, p0["W1a"], p1["W1a"])
    agg = _sc_segsum(y.reshape(2 * NPAD, DP), src, dst,
                     zeros).reshape(2, NPAD, DP)
    y = _round(y, agg, rw(p0, 1), rw(p1, 1))
    agg = _sc_segsum(y.reshape(2 * NPAD, DP), src, dst,
                     zeros).reshape(2, NPAD, DP)
    y = _round(y, agg, rw(p0, 2), rw(p1, 2))
    agg = _sc_segsum(y.reshape(2 * NPAD, DP), src, dst,
                     zeros).reshape(2, NPAD, DP)
    s, cnt = _pool(y, agg, pw(p0), pw(p1), bat0, bat1)
    wm = params["Wm"]
    return _head(s, cnt, wm[:DIM], wm[DIM:], params["bm"][None],
                 params["Wo"], params["bo"][None], params["Wf"],
                 params["bf"][None])
